# baseline (speedup 1.0000x reference)
import functools

import jax
import jax.numpy as jnp
from jax import lax
from jax.experimental import pallas as pl
from jax.experimental.pallas import tpu as pltpu
from jax.experimental.pallas import tpu_sc as plsc

VSZ = 1000000
DSZ = 64
PAD = 0

B = 16384 * 20
NC, NS = 2, 16
NW = NC * NS
CH = 512
CHUNKS = B // (NW * CH)
ROWS_W = B // NW
NSETS = 3


def _pad_fix(rows_ref, idx_ref, j):
    zeros = jnp.zeros((16,), jnp.float32)
    lanes = lax.iota(jnp.int32, 16)

    def body(t, carry):
        iv = idx_ref[j, pl.ds(t * 16, 16)]
        msk = iv == PAD
        row_ids = t * 16 + lanes
        for c in range(DSZ):
            col = jnp.full((16,), c, jnp.int32)
            plsc.store_scatter(rows_ref, [row_ids, col], zeros, mask=msk)
        return carry

    lax.fori_loop(0, CH // 16, body, 0)


def _make_gather():
    mesh = plsc.VectorSubcoreMesh(core_axis_name="c", subcore_axis_name="s")

    @functools.partial(
        pl.kernel,
        mesh=mesh,
        compiler_params=pltpu.CompilerParams(
            needs_layout_passes=False, use_tc_tiling_on_sc=False),
        out_type=jax.ShapeDtypeStruct((B, DSZ), jnp.float32),
        scratch_types=[
            pltpu.VMEM((CHUNKS, CH), jnp.int32),
            pltpu.VMEM((NSETS, CH, DSZ), jnp.float32),
            pltpu.SemaphoreType.DMA,
            pltpu.SemaphoreType.DMA,
            pltpu.SemaphoreType.DMA,
            pltpu.SemaphoreType.DMA,
            pltpu.SemaphoreType.DMA,
            pltpu.SemaphoreType.DMA,
        ],
    )
    def k(x_hbm, w_hbm, out_hbm, idx_v, rows_v,
          gsem0, gsem1, gsem2, osem0, osem1, osem2):
        wid = lax.axis_index("s") * NC + lax.axis_index("c")
        pltpu.sync_copy(x_hbm.at[pl.ds(wid * CHUNKS, CHUNKS)], idx_v)
        base = wid * ROWS_W
        gsems = (gsem0, gsem1, gsem2)
        osems = (osem0, osem1, osem2)

        def fire_gather(c, s):
            pltpu.async_copy(w_hbm.at[idx_v.at[c]], rows_v.at[s], gsems[s])

        def drain_g(s):
            pltpu.make_async_copy(
                w_hbm.at[idx_v.at[0]], rows_v.at[s], gsems[s]).wait()

        def drain_o(s):
            pltpu.make_async_copy(
                rows_v.at[s], out_hbm.at[pl.ds(0, CH)], osems[s]).wait()

        def do_chunk(c, s):
            o = (s + 2) % NSETS
            drain_g(s)
            _pad_fix(rows_v.at[s], idx_v, c)
            pltpu.async_copy(
                rows_v.at[s], out_hbm.at[pl.ds(base + c * CH, CH)], osems[s])

            @pl.when((c + 2 < CHUNKS) & (c >= 1))
            def _():
                drain_o(o)

            @pl.when(c + 2 < CHUNKS)
            def _():
                fire_gather(c + 2, o)

        fire_gather(0, 0)
        fire_gather(1, 1)

        def body(i, carry):
            c = NSETS * i
            do_chunk(c, 0)
            do_chunk(c + 1, 1)
            do_chunk(c + 2, 2)
            return carry

        lax.fori_loop(0, (CHUNKS - 2) // NSETS, body, 0)
        do_chunk(CHUNKS - 2, (CHUNKS - 2) % NSETS)
        do_chunk(CHUNKS - 1, (CHUNKS - 1) % NSETS)
        drain_o((CHUNKS - 3) % NSETS)
        drain_o((CHUNKS - 2) % NSETS)
        drain_o((CHUNKS - 1) % NSETS)

    return k


_gather = _make_gather()


def kernel(x, W):
    x2 = x.reshape(-1).astype(jnp.int32).reshape(NW * CHUNKS, CH)
    out = _gather(x2, W)
    return out.reshape(16384, 20, DSZ)


# speedup vs baseline: 1.0911x; 1.0911x over previous
import functools

import jax
import jax.numpy as jnp
from jax import lax
from jax.experimental import pallas as pl
from jax.experimental.pallas import tpu as pltpu
from jax.experimental.pallas import tpu_sc as plsc

VSZ = 1000000
DSZ = 64
PAD = 0

NI, NJ = 16384, 20
B = NI * NJ
NC, NS = 2, 16
NW = NC * NS
CH = 512
CHUNKS = B // (NW * CH)
SUB = CH // 128
NSETS = 3


def _pad_fix(rows_ref, idx_ref, j):
    zeros = jnp.zeros((16,), jnp.float32)
    lanes = lax.iota(jnp.int32, 16)

    def body(t, carry):
        iv = idx_ref[j, pl.ds(t * 16, 16)]
        msk = iv == PAD
        row_ids = t * 16 + lanes
        for c in range(DSZ):
            col = jnp.full((16,), c, jnp.int32)
            plsc.store_scatter(rows_ref, [row_ids, col], zeros, mask=msk)
        return carry

    lax.fori_loop(0, CH // 16, body, 0)


def _make_gather():
    mesh = plsc.VectorSubcoreMesh(core_axis_name="c", subcore_axis_name="s")

    @functools.partial(
        pl.kernel,
        mesh=mesh,
        compiler_params=pltpu.CompilerParams(
            needs_layout_passes=False, use_tc_tiling_on_sc=False),
        out_type=jax.ShapeDtypeStruct((NI, NJ * DSZ), jnp.float32),
        scratch_types=[
            pltpu.VMEM((CHUNKS, CH), jnp.int32),
            pltpu.VMEM((NSETS, CH, DSZ), jnp.float32),
            pltpu.SemaphoreType.DMA,
            pltpu.SemaphoreType.DMA,
            pltpu.SemaphoreType.DMA,
            pltpu.SemaphoreType.DMA,
            pltpu.SemaphoreType.DMA,
            pltpu.SemaphoreType.DMA,
        ],
    )
    def k(x_hbm, w_hbm, out_hbm, idx_v, rows_v,
          gsem0, gsem1, gsem2, osem0, osem1, osem2):
        wid = lax.axis_index("s") * NC + lax.axis_index("c")
        pltpu.sync_copy(x_hbm.at[pl.ds(wid * CHUNKS, CHUNKS)], idx_v)
        gsems = (gsem0, gsem1, gsem2)
        osems = (osem0, osem1, osem2)

        def fire_gather(c, s):
            pltpu.async_copy(w_hbm.at[idx_v.at[c]], rows_v.at[s], gsems[s])

        def drain_g(s):
            pltpu.make_async_copy(
                w_hbm.at[idx_v.at[0]], rows_v.at[s], gsems[s]).wait()

        def drain_o(s):
            for b in range(SUB):
                pltpu.make_async_copy(
                    rows_v.at[s, pl.ds(0, 128)],
                    out_hbm.at[pl.ds(0, 128), pl.ds(0, DSZ)],
                    osems[s]).wait()

        def do_chunk(c, s):
            o = (s + 2) % NSETS
            drain_g(s)
            _pad_fix(rows_v.at[s], idx_v, c)
            for b in range(SUB):
                r = (wid * CHUNKS + c) * SUB + b
                j = r // (NI // 128)
                i0 = (r % (NI // 128)) * 128
                pltpu.async_copy(
                    rows_v.at[s, pl.ds(b * 128, 128)],
                    out_hbm.at[pl.ds(i0, 128), pl.ds(j * DSZ, DSZ)],
                    osems[s])

            @pl.when((c + 2 < CHUNKS) & (c >= 1))
            def _():
                drain_o(o)

            @pl.when(c + 2 < CHUNKS)
            def _():
                fire_gather(c + 2, o)

        fire_gather(0, 0)
        fire_gather(1, 1)

        def body(i, carry):
            c = NSETS * i
            do_chunk(c, 0)
            do_chunk(c + 1, 1)
            do_chunk(c + 2, 2)
            return carry

        lax.fori_loop(0, (CHUNKS - 2) // NSETS, body, 0)
        do_chunk(CHUNKS - 2, (CHUNKS - 2) % NSETS)
        do_chunk(CHUNKS - 1, (CHUNKS - 1) % NSETS)
        drain_o((CHUNKS - 3) % NSETS)
        drain_o((CHUNKS - 2) % NSETS)
        drain_o((CHUNKS - 1) % NSETS)

    return k


_gather = _make_gather()


def kernel(x, W):
    x2 = jnp.swapaxes(x, 0, 1).astype(jnp.int32).reshape(NW * CHUNKS, CH)
    out = _gather(x2, W)
    return out.reshape(NI, NJ, DSZ)


# speedup vs baseline: 1.0914x; 1.0002x over previous
import functools

import jax
import jax.numpy as jnp
from jax import lax
from jax.experimental import pallas as pl
from jax.experimental.pallas import tpu as pltpu
from jax.experimental.pallas import tpu_sc as plsc

VSZ = 1000000
DSZ = 64
PAD = 0

NI, NJ = 16384, 20
B = NI * NJ
NC, NS = 2, 16
NW = NC * NS
CH = 512
CHUNKS = B // (NW * CH)
SUB = CH // 128
NSETS = 3


def _pad_fix(rows_ref, idx_ref, j):
    zeros = jnp.zeros((16,), jnp.float32)
    lanes = lax.iota(jnp.int32, 16)

    def or_body(t, acc):
        iv = idx_ref[j, pl.ds(t * 16, 16)]
        return acc | (iv == PAD)

    anymask = lax.fori_loop(
        0, CH // 16, or_body, jnp.zeros((16,), jnp.bool_))
    cnt = plsc.all_reduce_population_count(anymask)

    @pl.when(cnt[0] > 0)
    def _():
        def body(t, carry):
            iv = idx_ref[j, pl.ds(t * 16, 16)]
            msk = iv == PAD
            row_ids = t * 16 + lanes
            for c in range(DSZ):
                col = jnp.full((16,), c, jnp.int32)
                plsc.store_scatter(rows_ref, [row_ids, col], zeros, mask=msk)
            return carry

        lax.fori_loop(0, CH // 16, body, 0)


def _make_gather():
    mesh = plsc.VectorSubcoreMesh(core_axis_name="c", subcore_axis_name="s")

    @functools.partial(
        pl.kernel,
        mesh=mesh,
        compiler_params=pltpu.CompilerParams(
            needs_layout_passes=False, use_tc_tiling_on_sc=False),
        out_type=jax.ShapeDtypeStruct((NI, NJ * DSZ), jnp.float32),
        scratch_types=[
            pltpu.VMEM((CHUNKS, CH), jnp.int32),
            pltpu.VMEM((NSETS, CH, DSZ), jnp.float32),
            pltpu.SemaphoreType.DMA,
            pltpu.SemaphoreType.DMA,
            pltpu.SemaphoreType.DMA,
            pltpu.SemaphoreType.DMA,
            pltpu.SemaphoreType.DMA,
            pltpu.SemaphoreType.DMA,
        ],
    )
    def k(x_hbm, w_hbm, out_hbm, idx_v, rows_v,
          gsem0, gsem1, gsem2, osem0, osem1, osem2):
        wid = lax.axis_index("s") * NC + lax.axis_index("c")
        pltpu.sync_copy(x_hbm.at[pl.ds(wid * CHUNKS, CHUNKS)], idx_v)
        gsems = (gsem0, gsem1, gsem2)
        osems = (osem0, osem1, osem2)

        def fire_gather(c, s):
            pltpu.async_copy(w_hbm.at[idx_v.at[c]], rows_v.at[s], gsems[s])

        def drain_g(s):
            pltpu.make_async_copy(
                w_hbm.at[idx_v.at[0]], rows_v.at[s], gsems[s]).wait()

        def drain_o(s):
            for b in range(SUB):
                pltpu.make_async_copy(
                    rows_v.at[s, pl.ds(0, 128)],
                    out_hbm.at[pl.ds(0, 128), pl.ds(0, DSZ)],
                    osems[s]).wait()

        def do_chunk(c, s):
            o = (s + 2) % NSETS
            drain_g(s)
            _pad_fix(rows_v.at[s], idx_v, c)
            for b in range(SUB):
                r = (wid * CHUNKS + c) * SUB + b
                j = r // (NI // 128)
                i0 = (r % (NI // 128)) * 128
                pltpu.async_copy(
                    rows_v.at[s, pl.ds(b * 128, 128)],
                    out_hbm.at[pl.ds(i0, 128), pl.ds(j * DSZ, DSZ)],
                    osems[s])

            @pl.when((c + 2 < CHUNKS) & (c >= 1))
            def _():
                drain_o(o)

            @pl.when(c + 2 < CHUNKS)
            def _():
                fire_gather(c + 2, o)

        fire_gather(0, 0)
        fire_gather(1, 1)

        def body(i, carry):
            c = NSETS * i
            do_chunk(c, 0)
            do_chunk(c + 1, 1)
            do_chunk(c + 2, 2)
            return carry

        lax.fori_loop(0, (CHUNKS - 2) // NSETS, body, 0)
        do_chunk(CHUNKS - 2, (CHUNKS - 2) % NSETS)
        do_chunk(CHUNKS - 1, (CHUNKS - 1) % NSETS)
        drain_o((CHUNKS - 3) % NSETS)
        drain_o((CHUNKS - 2) % NSETS)
        drain_o((CHUNKS - 1) % NSETS)

    return k


_gather = _make_gather()


def kernel(x, W):
    x2 = jnp.swapaxes(x, 0, 1).astype(jnp.int32).reshape(NW * CHUNKS, CH)
    out = _gather(x2, W)
    return out.reshape(NI, NJ, DSZ)
